# Initial kernel scaffold; baseline (speedup 1.0000x reference)
import functools

import jax
import jax.numpy as jnp
from jax import lax
from jax.experimental import pallas as pl
from jax.experimental.pallas import tpu as pltpu
from jax.experimental.pallas import tpu_sc as plsc

N = 10000
E = 320000
D = 128
H = 128
C = 40

NC = 2
NS = 16
NW = NC * NS

LANES = 128
N_PAD = 10112
DUMMY = N_PAD - 1
ROWS_PER_SUB = N_PAD // NS
E_ROWS = 2528
ROWS_PER_W = E_ROWS // NW

_mesh = plsc.VectorSubcoreMesh(core_axis_name="c", subcore_axis_name="s")


@functools.partial(
    pl.kernel,
    out_type=jax.ShapeDtypeStruct((NC, N_PAD, 1), jnp.float32),
    mesh=_mesh,
    scratch_types=[
        pltpu.VMEM((ROWS_PER_W, LANES), jnp.int32),
        pltpu.VMEM((LANES, 1), jnp.float32),
        pltpu.VMEM_SHARED((N_PAD, 1), jnp.float32),
    ],
)
def _sc_degree(dst_hbm, ones_hbm, zeros1_hbm, out_hbm, dstv, onesv, acc):
    c = lax.axis_index("c")
    s = lax.axis_index("s")
    wid = c * NS + s
    pltpu.sync_copy(zeros1_hbm, acc.at[pl.ds(s * ROWS_PER_SUB, ROWS_PER_SUB)])
    pltpu.sync_copy(dst_hbm.at[pl.ds(wid * ROWS_PER_W, ROWS_PER_W)], dstv)
    pltpu.sync_copy(ones_hbm, onesv)
    plsc.subcore_barrier()

    @pl.loop(0, ROWS_PER_W)
    def _(j):
        pltpu.sync_copy(onesv, acc.at[dstv.at[j]], add=True)

    plsc.subcore_barrier()
    pltpu.sync_copy(
        acc.at[pl.ds(s * ROWS_PER_SUB, ROWS_PER_SUB)],
        out_hbm.at[c, pl.ds(s * ROWS_PER_SUB, ROWS_PER_SUB)],
    )


@functools.partial(
    pl.kernel,
    out_type=jax.ShapeDtypeStruct((NC, N_PAD, H), jnp.float32),
    mesh=_mesh,
    scratch_types=[
        pltpu.VMEM((ROWS_PER_W, LANES), jnp.int32),
        pltpu.VMEM((ROWS_PER_W, LANES), jnp.int32),
        pltpu.VMEM((LANES, H), jnp.float32),
        pltpu.VMEM_SHARED((N_PAD, H), jnp.float32),
        pltpu.SemaphoreType.DMA,
    ],
)
def _sc_scatter(g_hbm, src_hbm, dst_hbm, zeros_hbm, out_hbm,
                srcv, dstv, rowbuf, acc, sem):
    c = lax.axis_index("c")
    s = lax.axis_index("s")
    wid = c * NS + s
    pltpu.sync_copy(zeros_hbm, acc.at[pl.ds(s * ROWS_PER_SUB, ROWS_PER_SUB)])
    pltpu.sync_copy(src_hbm.at[pl.ds(wid * ROWS_PER_W, ROWS_PER_W)], srcv)
    pltpu.sync_copy(dst_hbm.at[pl.ds(wid * ROWS_PER_W, ROWS_PER_W)], dstv)
    plsc.subcore_barrier()

    @pl.loop(0, ROWS_PER_W)
    def _(j):
        pltpu.async_copy(g_hbm.at[srcv.at[j]], rowbuf, sem).wait()
        pltpu.sync_copy(rowbuf, acc.at[dstv.at[j]], add=True)

    plsc.subcore_barrier()
    pltpu.sync_copy(
        acc.at[pl.ds(s * ROWS_PER_SUB, ROWS_PER_SUB)],
        out_hbm.at[c, pl.ds(s * ROWS_PER_SUB, ROWS_PER_SUB)],
    )


BLK = 1264
GRID = N_PAD // BLK


def _tc_mm(x, W):
    def body(x_ref, w_ref, o_ref):
        o_ref[...] = jnp.dot(x_ref[...], w_ref[...],
                             preferred_element_type=jnp.float32)
    return pl.pallas_call(
        body,
        grid=(GRID,),
        in_specs=[pl.BlockSpec((BLK, D), lambda i: (i, 0)),
                  pl.BlockSpec((D, H), lambda i: (0, 0))],
        out_specs=pl.BlockSpec((BLK, H), lambda i: (i, 0)),
        out_shape=jax.ShapeDtypeStruct((N_PAD, H), jnp.float32),
    )(x, W)


def _tc_prep(degp, m1):
    def body(deg_ref, m_ref, dinv_ref, g_ref):
        dinv = lax.rsqrt(deg_ref[0] + deg_ref[1] + 1.0)
        dinv_ref[...] = dinv
        g_ref[...] = m_ref[...] * dinv
    return pl.pallas_call(
        body,
        grid=(GRID,),
        in_specs=[pl.BlockSpec((NC, BLK, 1), lambda i: (0, i, 0)),
                  pl.BlockSpec((BLK, H), lambda i: (i, 0))],
        out_specs=[pl.BlockSpec((BLK, 1), lambda i: (i, 0)),
                   pl.BlockSpec((BLK, H), lambda i: (i, 0))],
        out_shape=[jax.ShapeDtypeStruct((N_PAD, 1), jnp.float32),
                   jax.ShapeDtypeStruct((N_PAD, H), jnp.float32)],
    )(degp, m1)


def _tc_layer(parts, g_prev, dinv, b, W_next):
    def body(p_ref, g_ref, dinv_ref, b_ref, w_ref, o_ref):
        ssum = p_ref[0] + p_ref[1] + g_ref[...]
        h = jnp.tanh(dinv_ref[...] * ssum + b_ref[...])
        o_ref[...] = jnp.dot(h, w_ref[...],
                             preferred_element_type=jnp.float32) * dinv_ref[...]
    return pl.pallas_call(
        body,
        grid=(GRID,),
        in_specs=[pl.BlockSpec((NC, BLK, H), lambda i: (0, i, 0)),
                  pl.BlockSpec((BLK, H), lambda i: (i, 0)),
                  pl.BlockSpec((BLK, 1), lambda i: (i, 0)),
                  pl.BlockSpec((1, H), lambda i: (0, 0)),
                  pl.BlockSpec((H, H), lambda i: (0, 0))],
        out_specs=pl.BlockSpec((BLK, H), lambda i: (i, 0)),
        out_shape=jax.ShapeDtypeStruct((N_PAD, H), jnp.float32),
    )(parts, g_prev, dinv, b, W_next)


def _tc_final(parts, g_prev, dinv, b, Wc_pad, bc_pad):
    def body(p_ref, g_ref, dinv_ref, b_ref, wc_ref, bc_ref, h_ref, o_ref):
        ssum = p_ref[0] + p_ref[1] + g_ref[...]
        h = jnp.tanh(dinv_ref[...] * ssum + b_ref[...])
        h_ref[...] = h
        o_ref[...] = jnp.dot(h, wc_ref[...],
                             preferred_element_type=jnp.float32) + bc_ref[...]
    return pl.pallas_call(
        body,
        grid=(GRID,),
        in_specs=[pl.BlockSpec((NC, BLK, H), lambda i: (0, i, 0)),
                  pl.BlockSpec((BLK, H), lambda i: (i, 0)),
                  pl.BlockSpec((BLK, 1), lambda i: (i, 0)),
                  pl.BlockSpec((1, H), lambda i: (0, 0)),
                  pl.BlockSpec((H, H), lambda i: (0, 0)),
                  pl.BlockSpec((1, H), lambda i: (0, 0))],
        out_specs=[pl.BlockSpec((BLK, H), lambda i: (i, 0)),
                   pl.BlockSpec((BLK, H), lambda i: (i, 0))],
        out_shape=[jax.ShapeDtypeStruct((N_PAD, H), jnp.float32),
                   jax.ShapeDtypeStruct((N_PAD, H), jnp.float32)],
    )(parts, g_prev, dinv, b, Wc_pad, bc_pad)


@jax.jit
def kernel(x, edge_index, W1, b1, W2, b2, W3, b3, Wc, bc):
    f32 = jnp.float32
    x_pad = jnp.zeros((N_PAD, D), f32).at[:N].set(x)

    e_pad = E_ROWS * LANES - E
    src = jnp.concatenate([edge_index[0], jnp.zeros((e_pad,), jnp.int32)])
    dst = jnp.concatenate(
        [edge_index[1], jnp.full((e_pad,), DUMMY, jnp.int32)])
    src = src.reshape(E_ROWS, LANES)
    dst = dst.reshape(E_ROWS, LANES)

    ones_hbm = jnp.ones((LANES, 1), f32)
    zeros1_hbm = jnp.zeros((ROWS_PER_SUB, 1), f32)
    zeros_hbm = jnp.zeros((ROWS_PER_SUB, H), f32)

    Wc_pad = jnp.zeros((H, H), f32).at[:, :C].set(Wc)
    bc_pad = jnp.zeros((1, H), f32).at[0, :C].set(bc)
    b1r = b1.reshape(1, H)
    b2r = b2.reshape(1, H)
    b3r = b3.reshape(1, H)

    degp = _sc_degree(dst, ones_hbm, zeros1_hbm)
    m1 = _tc_mm(x_pad, W1)
    dinv, g1 = _tc_prep(degp, m1)

    s1 = _sc_scatter(g1, src, dst, zeros_hbm)
    g2 = _tc_layer(s1, g1, dinv, b1r, W2)
    s2 = _sc_scatter(g2, src, dst, zeros_hbm)
    g3 = _tc_layer(s2, g2, dinv, b2r, W3)
    s3 = _sc_scatter(g3, src, dst, zeros_hbm)
    h_pad, out_pad = _tc_final(s3, g3, dinv, b3r, Wc_pad, bc_pad)

    return out_pad[:N, :C], h_pad[:N]


# speedup vs baseline: 7.0861x; 7.0861x over previous
import functools

import jax
import jax.numpy as jnp
from jax import lax
from jax.experimental import pallas as pl
from jax.experimental.pallas import tpu as pltpu
from jax.experimental.pallas import tpu_sc as plsc

N = 10000
E = 320000
D = 128
H = 128
C = 40

NC = 2
NS = 16
NW = NC * NS

LANES = 128
N_PAD = 10112
DUMMY = N_PAD - 1
ROWS_PER_SUB = N_PAD // NS
E_ROWS = 2560
ROWS_PER_W = E_ROWS // NW

_mesh = plsc.VectorSubcoreMesh(core_axis_name="c", subcore_axis_name="s")


@functools.partial(
    pl.kernel,
    out_type=jax.ShapeDtypeStruct((NC, N_PAD, H), jnp.float32),
    mesh=_mesh,
    scratch_types=[
        pltpu.VMEM((ROWS_PER_W, LANES), jnp.int32),
        pltpu.VMEM((LANES, H), jnp.float32),
        pltpu.VMEM_SHARED((N_PAD, H), jnp.float32),
    ],
)
def _sc_degree(dst_hbm, ones_hbm, zeros_hbm, out_hbm, dstv, onesv, acc):
    c = lax.axis_index("c")
    s = lax.axis_index("s")
    wid = c * NS + s
    pltpu.sync_copy(zeros_hbm, acc.at[pl.ds(s * ROWS_PER_SUB, ROWS_PER_SUB)])
    pltpu.sync_copy(dst_hbm.at[pl.ds(wid * ROWS_PER_W, ROWS_PER_W)], dstv)
    pltpu.sync_copy(ones_hbm, onesv)
    plsc.subcore_barrier()

    @pl.loop(0, ROWS_PER_W)
    def _(j):
        pltpu.sync_copy(onesv, acc.at[dstv.at[j]], add=True)

    plsc.subcore_barrier()
    pltpu.sync_copy(
        acc.at[pl.ds(s * ROWS_PER_SUB, ROWS_PER_SUB)],
        out_hbm.at[c, pl.ds(s * ROWS_PER_SUB, ROWS_PER_SUB)],
    )


@functools.partial(
    pl.kernel,
    out_type=jax.ShapeDtypeStruct((NC, N_PAD, H), jnp.float32),
    mesh=_mesh,
    scratch_types=[
        pltpu.VMEM((ROWS_PER_W, LANES), jnp.int32),
        pltpu.VMEM((ROWS_PER_W, LANES), jnp.int32),
        pltpu.VMEM((LANES, H), jnp.float32),
        pltpu.VMEM_SHARED((N_PAD, H), jnp.float32),
        pltpu.SemaphoreType.DMA,
    ],
)
def _sc_scatter(g_hbm, src_hbm, dst_hbm, zeros_hbm, out_hbm,
                srcv, dstv, rowbuf, acc, sem):
    c = lax.axis_index("c")
    s = lax.axis_index("s")
    wid = c * NS + s
    pltpu.sync_copy(zeros_hbm, acc.at[pl.ds(s * ROWS_PER_SUB, ROWS_PER_SUB)])
    pltpu.sync_copy(src_hbm.at[pl.ds(wid * ROWS_PER_W, ROWS_PER_W)], srcv)
    pltpu.sync_copy(dst_hbm.at[pl.ds(wid * ROWS_PER_W, ROWS_PER_W)], dstv)
    plsc.subcore_barrier()

    @pl.loop(0, ROWS_PER_W)
    def _(j):
        pltpu.async_copy(g_hbm.at[srcv.at[j]], rowbuf, sem).wait()
        pltpu.sync_copy(rowbuf, acc.at[dstv.at[j]], add=True)

    plsc.subcore_barrier()
    pltpu.sync_copy(
        acc.at[pl.ds(s * ROWS_PER_SUB, ROWS_PER_SUB)],
        out_hbm.at[c, pl.ds(s * ROWS_PER_SUB, ROWS_PER_SUB)],
    )


BLK = 1264
GRID = N_PAD // BLK


def _tc_mm(x, W):
    def body(x_ref, w_ref, o_ref):
        o_ref[...] = jnp.dot(x_ref[...], w_ref[...],
                             preferred_element_type=jnp.float32)
    return pl.pallas_call(
        body,
        grid=(GRID,),
        in_specs=[pl.BlockSpec((BLK, D), lambda i: (i, 0)),
                  pl.BlockSpec((D, H), lambda i: (0, 0))],
        out_specs=pl.BlockSpec((BLK, H), lambda i: (i, 0)),
        out_shape=jax.ShapeDtypeStruct((N_PAD, H), jnp.float32),
    )(x, W)


def _tc_prep(degp, m1):
    def body(deg_ref, m_ref, dinv_ref, g_ref):
        dinv = lax.rsqrt(deg_ref[0, :, 0:1] + deg_ref[1, :, 0:1] + 1.0)
        dinv_ref[...] = dinv
        g_ref[...] = m_ref[...] * dinv
    return pl.pallas_call(
        body,
        grid=(GRID,),
        in_specs=[pl.BlockSpec((NC, BLK, H), lambda i: (0, i, 0)),
                  pl.BlockSpec((BLK, H), lambda i: (i, 0))],
        out_specs=[pl.BlockSpec((BLK, 1), lambda i: (i, 0)),
                   pl.BlockSpec((BLK, H), lambda i: (i, 0))],
        out_shape=[jax.ShapeDtypeStruct((N_PAD, 1), jnp.float32),
                   jax.ShapeDtypeStruct((N_PAD, H), jnp.float32)],
    )(degp, m1)


def _tc_layer(parts, g_prev, dinv, b, W_next):
    def body(p_ref, g_ref, dinv_ref, b_ref, w_ref, o_ref):
        ssum = p_ref[0] + p_ref[1] + g_ref[...]
        h = jnp.tanh(dinv_ref[...] * ssum + b_ref[...])
        o_ref[...] = jnp.dot(h, w_ref[...],
                             preferred_element_type=jnp.float32) * dinv_ref[...]
    return pl.pallas_call(
        body,
        grid=(GRID,),
        in_specs=[pl.BlockSpec((NC, BLK, H), lambda i: (0, i, 0)),
                  pl.BlockSpec((BLK, H), lambda i: (i, 0)),
                  pl.BlockSpec((BLK, 1), lambda i: (i, 0)),
                  pl.BlockSpec((1, H), lambda i: (0, 0)),
                  pl.BlockSpec((H, H), lambda i: (0, 0))],
        out_specs=pl.BlockSpec((BLK, H), lambda i: (i, 0)),
        out_shape=jax.ShapeDtypeStruct((N_PAD, H), jnp.float32),
    )(parts, g_prev, dinv, b, W_next)


def _tc_final(parts, g_prev, dinv, b, Wc_pad, bc_pad):
    def body(p_ref, g_ref, dinv_ref, b_ref, wc_ref, bc_ref, h_ref, o_ref):
        ssum = p_ref[0] + p_ref[1] + g_ref[...]
        h = jnp.tanh(dinv_ref[...] * ssum + b_ref[...])
        h_ref[...] = h
        o_ref[...] = jnp.dot(h, wc_ref[...],
                             preferred_element_type=jnp.float32) + bc_ref[...]
    return pl.pallas_call(
        body,
        grid=(GRID,),
        in_specs=[pl.BlockSpec((NC, BLK, H), lambda i: (0, i, 0)),
                  pl.BlockSpec((BLK, H), lambda i: (i, 0)),
                  pl.BlockSpec((BLK, 1), lambda i: (i, 0)),
                  pl.BlockSpec((1, H), lambda i: (0, 0)),
                  pl.BlockSpec((H, H), lambda i: (0, 0)),
                  pl.BlockSpec((1, H), lambda i: (0, 0))],
        out_specs=[pl.BlockSpec((BLK, H), lambda i: (i, 0)),
                   pl.BlockSpec((BLK, H), lambda i: (i, 0))],
        out_shape=[jax.ShapeDtypeStruct((N_PAD, H), jnp.float32),
                   jax.ShapeDtypeStruct((N_PAD, H), jnp.float32)],
    )(parts, g_prev, dinv, b, Wc_pad, bc_pad)


@jax.jit
def kernel(x, edge_index, W1, b1, W2, b2, W3, b3, Wc, bc):
    f32 = jnp.float32
    x_pad = jnp.zeros((N_PAD, D), f32).at[:N].set(x)

    e_pad = E_ROWS * LANES - E
    src = jnp.concatenate([edge_index[0], jnp.zeros((e_pad,), jnp.int32)])
    dst = jnp.concatenate(
        [edge_index[1], jnp.full((e_pad,), DUMMY, jnp.int32)])
    src = src.reshape(E_ROWS, LANES)
    dst = dst.reshape(E_ROWS, LANES)

    ones_hbm = jnp.ones((LANES, H), f32)
    zeros_hbm = jnp.zeros((ROWS_PER_SUB, H), f32)

    Wc_pad = jnp.zeros((H, H), f32).at[:, :C].set(Wc)
    bc_pad = jnp.zeros((1, H), f32).at[0, :C].set(bc)
    b1r = b1.reshape(1, H)
    b2r = b2.reshape(1, H)
    b3r = b3.reshape(1, H)

    degp = _sc_degree(dst, ones_hbm, zeros_hbm)
    m1 = _tc_mm(x_pad, W1)
    dinv, g1 = _tc_prep(degp, m1)

    s1 = _sc_scatter(g1, src, dst, zeros_hbm)
    g2 = _tc_layer(s1, g1, dinv, b1r, W2)
    s2 = _sc_scatter(g2, src, dst, zeros_hbm)
    g3 = _tc_layer(s2, g2, dinv, b2r, W3)
    s3 = _sc_scatter(g3, src, dst, zeros_hbm)
    h_pad, out_pad = _tc_final(s3, g3, dinv, b3r, Wc_pad, bc_pad)

    return out_pad[:N, :C], h_pad[:N]
